# Initial kernel scaffold; baseline (speedup 1.0000x reference)
import jax
import jax.numpy as jnp
from jax.experimental import pallas as pl


def kernel(x):
    raise NotImplementedError("write your pallas kernel here")


# speedup vs baseline: 1.7904x; 1.7904x over previous
import functools

import jax
import jax.numpy as jnp
from jax import lax
from jax.experimental import pallas as pl
from jax.experimental.pallas import tpu as pltpu
from jax.experimental.pallas import tpu_sc as plsc

_N = 16384
_D = 128
_NC = 2
_NS = 16
_LANES = 16
_RADIX = 256


_P0 = (-5.99633501014107895267E1, 9.80010754185999661536E1,
       -5.66762857469070293439E1, 1.39312609387279679503E1,
       -1.23916583867381258016E0)
_Q0 = (1.0, 1.95448858338141759834E0, 4.67627912898881538453E0,
       8.63602421390890590575E1, -2.25462687854119370527E2,
       2.00260212380060660359E2, -8.20372256168333339912E1,
       1.59056225126211695515E1, -1.18331621121330003142E0)
_P1 = (4.05544892305962419923E0, 3.15251094599893866154E1,
       5.71628192246421288162E1, 4.40805073893200834700E1,
       1.46849561928858024014E1, 2.18663306850790267539E0,
       -1.40256079171354495875E-1, -3.50424626827848203418E-2,
       -8.57456785154685413611E-4)
_Q1 = (1.0, 1.57799883256466749731E1, 4.53907635128879210584E1,
       4.13172038254672030440E1, 1.50425385692907503408E1,
       2.50464946208309415979E0, -1.42182922854787788574E-1,
       -3.80806407691578277194E-2, -9.33259480895457427372E-4)
_P2 = (3.23774891776946035970E0, 6.91522889068984211695E0,
       3.93881025292474443415E0, 1.33303460815807542389E0,
       2.01485389549179081538E-1, 1.23716634817820021358E-2,
       3.01581553508235416007E-4, 2.65806974686737550832E-6,
       6.23974539184983293730E-9)
_Q2 = (1.0, 6.02427039364742014255E0, 3.67983563856160859403E0,
       1.37702099489081330271E0, 2.16236993594496635890E-1,
       1.34204006088543189037E-2, 3.28014464682127739104E-4,
       2.89247864745380683936E-6, 6.79019408009981274425E-9)
_EXPM2 = 0.1353352832366127
_S2PI = 2.5066282746310002


def _polyval(coefs, x):
    r = jnp.full_like(x, coefs[0])
    for c in coefs[1:]:
        r = r * x + c
    return r


def _ndtri(p):
    mcp = jnp.where(p > 1.0 - _EXPM2, 1.0 - p, p)
    w = mcp - 0.5
    ww = w * w
    big = (w + w * ww * (_polyval(_P0, ww) / _polyval(_Q0, ww))) * (-_S2PI)
    z = jnp.sqrt(-2.0 * jnp.log(mcp))
    ft = z - jnp.log(z) / z
    zi = 1.0 / z
    small = ft - _polyval(_P2, zi) / _polyval(_Q2, zi) * zi
    other = ft - _polyval(_P1, zi) / _polyval(_Q1, zi) * zi
    x = jnp.where(mcp > _EXPM2, big, jnp.where(z >= 8.0, small, other))
    return jnp.where(p > 1.0 - _EXPM2, x, -x)


def _icdf_table(n, side):
    def body(q_ref):
        r = lax.broadcasted_iota(jnp.int32, (side, side), 0)
        c = lax.broadcasted_iota(jnp.int32, (side, side), 1)
        k = (r * side + c).astype(jnp.float32)
        u = (k + 1.0) / float(n + 1)
        q_ref[...] = _ndtri(u)

    q = pl.pallas_call(
        body, out_shape=jax.ShapeDtypeStruct((side, side), jnp.float32))()
    return q.reshape(n)


def _build_rank_kernel(n, d, nc=_NC, ns=_NS, interpret=False):
    nw = nc * ns
    cpw = d // nw
    vregs = n // _LANES
    hist_sz = _RADIX * _LANES

    mesh = plsc.VectorSubcoreMesh(
        core_axis_name="c", subcore_axis_name="s",
        num_cores=nc, num_subcores=ns)

    @functools.partial(
        pl.kernel,
        out_type=jax.ShapeDtypeStruct((d, n), jnp.float32),
        mesh=mesh,
        interpret=interpret,
        compiler_params=pltpu.CompilerParams(needs_layout_passes=False),
        scratch_types=[
            pltpu.VMEM((n,), jnp.float32),
            pltpu.VMEM((n,), jnp.int32),
            pltpu.VMEM((n,), jnp.int32),
            pltpu.VMEM((n,), jnp.int32),
            pltpu.VMEM((n,), jnp.int32),
            pltpu.VMEM((hist_sz,), jnp.int32),
            pltpu.VMEM((n,), jnp.float32),
            pltpu.VMEM((n,), jnp.float32),
        ],
    )
    def body(xt_hbm, q_hbm, out_hbm, kf, k1, p1, k2, p2, hist, qv, outv):
        wid = lax.axis_index("s") * nc + lax.axis_index("c")
        lane = lax.iota(jnp.int32, _LANES)
        lane_s = lane * vregs
        ones = jnp.ones((_LANES,), jnp.int32)

        pltpu.sync_copy(q_hbm, qv)

        def digit(k, shift):
            d_ = k if shift == 0 else (k >> shift)
            d_ = d_ & (_RADIX - 1)
            if shift == 24:
                d_ = d_ ^ 128
            return d_

        def to_key(f):
            b = plsc.bitcast(f, jnp.int32)
            return b ^ ((b >> 31) & jnp.int32(0x7FFFFFFF))

        def zero_hist():
            def zb(j, c):
                hist[pl.ds(j * _LANES, _LANES)] = jnp.zeros(
                    (_LANES,), jnp.int32)
                return c
            lax.fori_loop(0, _RADIX, zb, 0)

        def scan_hist():
            def sb(j, carry):
                h = hist[pl.ds(j * _LANES, _LANES)]
                s = plsc.cumsum(h)
                hist[pl.ds(j * _LANES, _LANES)] = s - h + carry
                return carry + jnp.sum(h)
            lax.fori_loop(0, _RADIX, sb, jnp.int32(0))

        def count_pass(load_key, shift):
            def hb(j, c):
                k = load_key(j)
                hidx = digit(k, shift) * _LANES + lane
                plsc.addupdate_scatter(hist, [hidx], ones)
                return c
            lax.fori_loop(0, vregs, hb, 0)

        for t in range(cpw):
            col = wid * cpw + t
            pltpu.sync_copy(xt_hbm.at[col], kf)

            zero_hist()
            count_pass(lambda j: to_key(plsc.load_gather(kf, [lane_s + j])), 0)
            scan_hist()

            def m1(j, c):
                idx = lane_s + j
                k = to_key(plsc.load_gather(kf, [idx]))
                hidx = digit(k, 0) * _LANES + lane
                ofs = plsc.load_gather(hist, [hidx])
                plsc.store_scatter(k1, [ofs], k)
                plsc.store_scatter(p1, [ofs], idx)
                plsc.store_scatter(hist, [hidx], ofs + 1)
                return c
            lax.fori_loop(0, vregs, m1, 0)

            for shift, ksrc, psrc, kdst, pdst in (
                    (8, k1, p1, k2, p2), (16, k2, p2, k1, p1)):
                zero_hist()
                count_pass(
                    functools.partial(
                        lambda j, ksrc: plsc.load_gather(ksrc, [lane_s + j]),
                        ksrc=ksrc),
                    shift)
                scan_hist()

                def mb(j, c, shift=shift, ksrc=ksrc, psrc=psrc,
                       kdst=kdst, pdst=pdst):
                    idx = lane_s + j
                    k = plsc.load_gather(ksrc, [idx])
                    p = plsc.load_gather(psrc, [idx])
                    hidx = digit(k, shift) * _LANES + lane
                    ofs = plsc.load_gather(hist, [hidx])
                    plsc.store_scatter(kdst, [ofs], k)
                    plsc.store_scatter(pdst, [ofs], p)
                    plsc.store_scatter(hist, [hidx], ofs + 1)
                    return c
                lax.fori_loop(0, vregs, mb, 0)

            zero_hist()
            count_pass(lambda j: plsc.load_gather(k1, [lane_s + j]), 24)
            scan_hist()

            def m4(j, c):
                idx = lane_s + j
                k = plsc.load_gather(k1, [idx])
                p = plsc.load_gather(p1, [idx])
                hidx = digit(k, 24) * _LANES + lane
                ofs = plsc.load_gather(hist, [hidx])
                z = plsc.load_gather(qv, [ofs])
                plsc.store_scatter(outv, [p], z)
                plsc.store_scatter(hist, [hidx], ofs + 1)
                return c
            lax.fori_loop(0, vregs, m4, 0)

            pltpu.sync_copy(outv, out_hbm.at[col])

    return body


def kernel(x):
    q = _icdf_table(_N, _D)
    xt = x.T
    zt = _build_rank_kernel(_N, _D)(xt, q)
    return zt.T


# speedup vs baseline: 1.8324x; 1.0235x over previous
import functools

import jax
import jax.numpy as jnp
from jax import lax
from jax.experimental import pallas as pl
from jax.experimental.pallas import tpu as pltpu
from jax.experimental.pallas import tpu_sc as plsc

_N = 16384
_D = 128
_NC = 2
_NS = 16
_LANES = 16
_RADIX = 256


_P0 = (-5.99633501014107895267E1, 9.80010754185999661536E1,
       -5.66762857469070293439E1, 1.39312609387279679503E1,
       -1.23916583867381258016E0)
_Q0 = (1.0, 1.95448858338141759834E0, 4.67627912898881538453E0,
       8.63602421390890590575E1, -2.25462687854119370527E2,
       2.00260212380060660359E2, -8.20372256168333339912E1,
       1.59056225126211695515E1, -1.18331621121330003142E0)
_P1 = (4.05544892305962419923E0, 3.15251094599893866154E1,
       5.71628192246421288162E1, 4.40805073893200834700E1,
       1.46849561928858024014E1, 2.18663306850790267539E0,
       -1.40256079171354495875E-1, -3.50424626827848203418E-2,
       -8.57456785154685413611E-4)
_Q1 = (1.0, 1.57799883256466749731E1, 4.53907635128879210584E1,
       4.13172038254672030440E1, 1.50425385692907503408E1,
       2.50464946208309415979E0, -1.42182922854787788574E-1,
       -3.80806407691578277194E-2, -9.33259480895457427372E-4)
_P2 = (3.23774891776946035970E0, 6.91522889068984211695E0,
       3.93881025292474443415E0, 1.33303460815807542389E0,
       2.01485389549179081538E-1, 1.23716634817820021358E-2,
       3.01581553508235416007E-4, 2.65806974686737550832E-6,
       6.23974539184983293730E-9)
_Q2 = (1.0, 6.02427039364742014255E0, 3.67983563856160859403E0,
       1.37702099489081330271E0, 2.16236993594496635890E-1,
       1.34204006088543189037E-2, 3.28014464682127739104E-4,
       2.89247864745380683936E-6, 6.79019408009981274425E-9)
_EXPM2 = 0.1353352832366127
_S2PI = 2.5066282746310002


def _polyval(coefs, x):
    r = jnp.full_like(x, coefs[0])
    for c in coefs[1:]:
        r = r * x + c
    return r


def _ndtri(p):
    mcp = jnp.where(p > 1.0 - _EXPM2, 1.0 - p, p)
    w = mcp - 0.5
    ww = w * w
    big = (w + w * ww * (_polyval(_P0, ww) / _polyval(_Q0, ww))) * (-_S2PI)
    z = jnp.sqrt(-2.0 * jnp.log(mcp))
    ft = z - jnp.log(z) / z
    zi = 1.0 / z
    small = ft - _polyval(_P2, zi) / _polyval(_Q2, zi) * zi
    other = ft - _polyval(_P1, zi) / _polyval(_Q1, zi) * zi
    x = jnp.where(mcp > _EXPM2, big, jnp.where(z >= 8.0, small, other))
    return jnp.where(p > 1.0 - _EXPM2, x, -x)


def _icdf_table(n, side):
    def body(q_ref):
        r = lax.broadcasted_iota(jnp.int32, (side, side), 0)
        c = lax.broadcasted_iota(jnp.int32, (side, side), 1)
        k = (r * side + c).astype(jnp.float32)
        u = (k + 1.0) / float(n + 1)
        q_ref[...] = _ndtri(u)

    q = pl.pallas_call(
        body, out_shape=jax.ShapeDtypeStruct((side, side), jnp.float32))()
    return q.reshape(n)


def _build_rank_kernel(n, d, nc=_NC, ns=_NS, interpret=False):
    nw = nc * ns
    cpw = d // nw
    vregs = n // _LANES
    hist_sz = _RADIX * _LANES

    mesh = plsc.VectorSubcoreMesh(
        core_axis_name="c", subcore_axis_name="s",
        num_cores=nc, num_subcores=ns)

    @functools.partial(
        pl.kernel,
        out_type=jax.ShapeDtypeStruct((d, n), jnp.float32),
        mesh=mesh,
        interpret=interpret,
        compiler_params=pltpu.CompilerParams(needs_layout_passes=False),
        scratch_types=[
            pltpu.VMEM((n,), jnp.float32),
            pltpu.VMEM((n,), jnp.int32),
            pltpu.VMEM((n,), jnp.int32),
            pltpu.VMEM((n,), jnp.int32),
            pltpu.VMEM((n,), jnp.int32),
            pltpu.VMEM((hist_sz,), jnp.int32),
            pltpu.VMEM((n,), jnp.float32),
            pltpu.VMEM((n,), jnp.float32),
        ],
    )
    def body(xt_hbm, q_hbm, out_hbm, kf, k1, p1, k2, p2, hist, qv, outv):
        wid = lax.axis_index("s") * nc + lax.axis_index("c")
        lane = lax.iota(jnp.int32, _LANES)
        lane_s = lane * vregs
        ones = jnp.ones((_LANES,), jnp.int32)

        pltpu.sync_copy(q_hbm, qv)

        def digit(k, shift):
            d_ = k if shift == 0 else (k >> shift)
            d_ = d_ & (_RADIX - 1)
            if shift == 24:
                d_ = d_ ^ 128
            return d_

        def to_key(f):
            b = plsc.bitcast(f, jnp.int32)
            return b ^ ((b >> 31) & jnp.int32(0x7FFFFFFF))

        def zero_hist():
            def zb(j, c):
                hist[pl.ds(j * _LANES, _LANES)] = jnp.zeros(
                    (_LANES,), jnp.int32)
                return c
            lax.fori_loop(0, _RADIX, zb, 0, unroll=8)

        def scan_hist():
            def sb(j, carry):
                h = hist[pl.ds(j * _LANES, _LANES)]
                s = plsc.cumsum(h)
                hist[pl.ds(j * _LANES, _LANES)] = s - h + carry
                return carry + jnp.sum(h)
            lax.fori_loop(0, _RADIX, sb, jnp.int32(0), unroll=4)

        def count_pass(load_key, shift):
            def hb(j, c):
                k = load_key(j)
                hidx = digit(k, shift) * _LANES + lane
                plsc.addupdate_scatter(hist, [hidx], ones)
                return c
            lax.fori_loop(0, vregs, hb, 0, unroll=8)

        for t in range(cpw):
            col = wid * cpw + t
            pltpu.sync_copy(xt_hbm.at[col], kf)

            zero_hist()
            count_pass(lambda j: to_key(plsc.load_gather(kf, [lane_s + j])), 0)
            scan_hist()

            def m1(j, c):
                idx = lane_s + j
                k = to_key(plsc.load_gather(kf, [idx]))
                hidx = digit(k, 0) * _LANES + lane
                ofs = plsc.load_gather(hist, [hidx])
                plsc.store_scatter(k1, [ofs], k)
                plsc.store_scatter(p1, [ofs], idx)
                plsc.store_scatter(hist, [hidx], ofs + 1)
                return c
            lax.fori_loop(0, vregs, m1, 0, unroll=4)

            for shift, ksrc, psrc, kdst, pdst in (
                    (8, k1, p1, k2, p2), (16, k2, p2, k1, p1)):
                zero_hist()
                count_pass(
                    functools.partial(
                        lambda j, ksrc: plsc.load_gather(ksrc, [lane_s + j]),
                        ksrc=ksrc),
                    shift)
                scan_hist()

                def mb(j, c, shift=shift, ksrc=ksrc, psrc=psrc,
                       kdst=kdst, pdst=pdst):
                    idx = lane_s + j
                    k = plsc.load_gather(ksrc, [idx])
                    p = plsc.load_gather(psrc, [idx])
                    hidx = digit(k, shift) * _LANES + lane
                    ofs = plsc.load_gather(hist, [hidx])
                    plsc.store_scatter(kdst, [ofs], k)
                    plsc.store_scatter(pdst, [ofs], p)
                    plsc.store_scatter(hist, [hidx], ofs + 1)
                    return c
                lax.fori_loop(0, vregs, mb, 0, unroll=4)

            zero_hist()
            count_pass(lambda j: plsc.load_gather(k1, [lane_s + j]), 24)
            scan_hist()

            def m4(j, c):
                idx = lane_s + j
                k = plsc.load_gather(k1, [idx])
                p = plsc.load_gather(p1, [idx])
                hidx = digit(k, 24) * _LANES + lane
                ofs = plsc.load_gather(hist, [hidx])
                z = plsc.load_gather(qv, [ofs])
                plsc.store_scatter(outv, [p], z)
                plsc.store_scatter(hist, [hidx], ofs + 1)
                return c
            lax.fori_loop(0, vregs, m4, 0, unroll=4)

            pltpu.sync_copy(outv, out_hbm.at[col])

    return body


def kernel(x):
    q = _icdf_table(_N, _D)
    xt = x.T
    zt = _build_rank_kernel(_N, _D)(xt, q)
    return zt.T


# speedup vs baseline: 3.1556x; 1.7221x over previous
import functools

import jax
import jax.numpy as jnp
from jax import lax
from jax.experimental import pallas as pl
from jax.experimental.pallas import tpu as pltpu
from jax.experimental.pallas import tpu_sc as plsc

_N = 16384
_D = 128
_NC = 2
_NS = 16
_LANES = 16
_RADIX = 256


_P0 = (-5.99633501014107895267E1, 9.80010754185999661536E1,
       -5.66762857469070293439E1, 1.39312609387279679503E1,
       -1.23916583867381258016E0)
_Q0 = (1.0, 1.95448858338141759834E0, 4.67627912898881538453E0,
       8.63602421390890590575E1, -2.25462687854119370527E2,
       2.00260212380060660359E2, -8.20372256168333339912E1,
       1.59056225126211695515E1, -1.18331621121330003142E0)
_P1 = (4.05544892305962419923E0, 3.15251094599893866154E1,
       5.71628192246421288162E1, 4.40805073893200834700E1,
       1.46849561928858024014E1, 2.18663306850790267539E0,
       -1.40256079171354495875E-1, -3.50424626827848203418E-2,
       -8.57456785154685413611E-4)
_Q1 = (1.0, 1.57799883256466749731E1, 4.53907635128879210584E1,
       4.13172038254672030440E1, 1.50425385692907503408E1,
       2.50464946208309415979E0, -1.42182922854787788574E-1,
       -3.80806407691578277194E-2, -9.33259480895457427372E-4)
_P2 = (3.23774891776946035970E0, 6.91522889068984211695E0,
       3.93881025292474443415E0, 1.33303460815807542389E0,
       2.01485389549179081538E-1, 1.23716634817820021358E-2,
       3.01581553508235416007E-4, 2.65806974686737550832E-6,
       6.23974539184983293730E-9)
_Q2 = (1.0, 6.02427039364742014255E0, 3.67983563856160859403E0,
       1.37702099489081330271E0, 2.16236993594496635890E-1,
       1.34204006088543189037E-2, 3.28014464682127739104E-4,
       2.89247864745380683936E-6, 6.79019408009981274425E-9)
_EXPM2 = 0.1353352832366127
_S2PI = 2.5066282746310002


def _polyval(coefs, x):
    r = jnp.full_like(x, coefs[0])
    for c in coefs[1:]:
        r = r * x + c
    return r


def _ndtri(p):
    mcp = jnp.where(p > 1.0 - _EXPM2, 1.0 - p, p)
    w = mcp - 0.5
    ww = w * w
    big = (w + w * ww * (_polyval(_P0, ww) / _polyval(_Q0, ww))) * (-_S2PI)
    z = jnp.sqrt(-2.0 * jnp.log(mcp))
    ft = z - jnp.log(z) / z
    zi = 1.0 / z
    small = ft - _polyval(_P2, zi) / _polyval(_Q2, zi) * zi
    other = ft - _polyval(_P1, zi) / _polyval(_Q1, zi) * zi
    x = jnp.where(mcp > _EXPM2, big, jnp.where(z >= 8.0, small, other))
    return jnp.where(p > 1.0 - _EXPM2, x, -x)


def _icdf_table(n, side):
    def body(q_ref):
        r = lax.broadcasted_iota(jnp.int32, (side, side), 0)
        c = lax.broadcasted_iota(jnp.int32, (side, side), 1)
        k = (r * side + c).astype(jnp.float32)
        u = (k + 1.0) / float(n + 1)
        q_ref[...] = _ndtri(u)

    q = pl.pallas_call(
        body, out_shape=jax.ShapeDtypeStruct((side, side), jnp.float32))()
    return q.reshape(n)


def _build_rank_kernel(n, d, nc=_NC, ns=_NS, interpret=False):
    nw = nc * ns
    cpw = d // nw
    vregs = n // _LANES
    hist_sz = _RADIX * _LANES

    mesh = plsc.VectorSubcoreMesh(
        core_axis_name="c", subcore_axis_name="s",
        num_cores=nc, num_subcores=ns)

    @functools.partial(
        pl.kernel,
        out_type=jax.ShapeDtypeStruct((d, n), jnp.float32),
        mesh=mesh,
        interpret=interpret,
        compiler_params=pltpu.CompilerParams(needs_layout_passes=False),
        scratch_types=[
            pltpu.VMEM((n,), jnp.float32),
            pltpu.VMEM((n + 16,), jnp.int32),
            pltpu.VMEM((n + 16,), jnp.int32),
            pltpu.VMEM((n + 16,), jnp.int32),
            pltpu.VMEM((n + 16,), jnp.int32),
            pltpu.VMEM((hist_sz,), jnp.int32),
            pltpu.VMEM((n,), jnp.float32),
            pltpu.VMEM((n,), jnp.float32),
        ],
    )
    def body(xt_hbm, q_hbm, out_hbm, kf, k1, p1, k2, p2, hist, qv, outv):
        wid = lax.axis_index("s") * nc + lax.axis_index("c")
        lane = lax.iota(jnp.int32, _LANES)
        lane_s = lane * vregs
        lane_sp = lane * (vregs + 1)
        ones = jnp.ones((_LANES,), jnp.int32)

        shv = vregs.bit_length() - 1

        def skew(pos):
            return pos + (pos >> shv)

        pltpu.sync_copy(q_hbm, qv)

        def digit(k, shift):
            d_ = k if shift == 0 else (k >> shift)
            d_ = d_ & (_RADIX - 1)
            if shift == 24:
                d_ = d_ ^ 128
            return d_

        def to_key(f):
            b = plsc.bitcast(f, jnp.int32)
            return b ^ ((b >> 31) & jnp.int32(0x7FFFFFFF))

        def zero_hist():
            def zb(j, c):
                hist[pl.ds(j * _LANES, _LANES)] = jnp.zeros(
                    (_LANES,), jnp.int32)
                return c
            lax.fori_loop(0, _RADIX, zb, 0, unroll=8)

        def scan_hist():
            def sb(j, carry):
                h = hist[pl.ds(j * _LANES, _LANES)]
                s = plsc.cumsum(h)
                hist[pl.ds(j * _LANES, _LANES)] = s - h + carry
                return carry + jnp.sum(h)
            lax.fori_loop(0, _RADIX, sb, jnp.int32(0), unroll=4)

        def count_pass(load_key, shift):
            def hb(j, c):
                k = load_key(j)
                hidx = digit(k, shift) * _LANES + lane
                plsc.addupdate_scatter(hist, [hidx], ones)
                return c
            lax.fori_loop(0, vregs, hb, 0, unroll=8)

        for t in range(cpw):
            col = wid * cpw + t
            pltpu.sync_copy(xt_hbm.at[col], kf)

            def r0(j, c):
                k = to_key(kf[pl.ds(j * _LANES, _LANES)])
                pos = j * _LANES + lane
                plsc.store_scatter(k2, [skew(pos)], k)
                return c
            lax.fori_loop(0, vregs, r0, 0, unroll=8)

            zero_hist()
            count_pass(lambda j: plsc.load_gather(k2, [lane_sp + j]), 0)
            scan_hist()

            def m1(j, c):
                k = plsc.load_gather(k2, [lane_sp + j])
                hidx = digit(k, 0) * _LANES + lane
                ofs = plsc.load_gather(hist, [hidx])
                fo = skew(ofs)
                plsc.store_scatter(k1, [fo], k)
                plsc.store_scatter(p1, [fo], lane_s + j)
                plsc.store_scatter(hist, [hidx], ofs + 1)
                return c
            lax.fori_loop(0, vregs, m1, 0, unroll=4)

            for shift, ksrc, psrc, kdst, pdst in (
                    (8, k1, p1, k2, p2), (16, k2, p2, k1, p1)):
                zero_hist()
                count_pass(
                    functools.partial(
                        lambda j, ksrc: plsc.load_gather(ksrc, [lane_sp + j]),
                        ksrc=ksrc),
                    shift)
                scan_hist()

                def mb(j, c, shift=shift, ksrc=ksrc, psrc=psrc,
                       kdst=kdst, pdst=pdst):
                    idx = lane_sp + j
                    k = plsc.load_gather(ksrc, [idx])
                    p = plsc.load_gather(psrc, [idx])
                    hidx = digit(k, shift) * _LANES + lane
                    ofs = plsc.load_gather(hist, [hidx])
                    fo = skew(ofs)
                    plsc.store_scatter(kdst, [fo], k)
                    plsc.store_scatter(pdst, [fo], p)
                    plsc.store_scatter(hist, [hidx], ofs + 1)
                    return c
                lax.fori_loop(0, vregs, mb, 0, unroll=4)

            zero_hist()
            count_pass(lambda j: plsc.load_gather(k1, [lane_sp + j]), 24)
            scan_hist()

            def m4(j, c):
                idx = lane_sp + j
                k = plsc.load_gather(k1, [idx])
                p = plsc.load_gather(p1, [idx])
                hidx = digit(k, 24) * _LANES + lane
                ofs = plsc.load_gather(hist, [hidx])
                z = plsc.load_gather(qv, [ofs])
                plsc.store_scatter(outv, [p], z)
                plsc.store_scatter(hist, [hidx], ofs + 1)
                return c
            lax.fori_loop(0, vregs, m4, 0, unroll=4)

            pltpu.sync_copy(outv, out_hbm.at[col])

    return body


def kernel(x):
    q = _icdf_table(_N, _D)
    xt = x.T
    zt = _build_rank_kernel(_N, _D)(xt, q)
    return zt.T
